# Initial kernel scaffold; baseline (speedup 1.0000x reference)
import jax
import jax.numpy as jnp
from jax.experimental import pallas as pl


def kernel(user_id, gender, age, occup, zip, year, month, hour, weekday, user_activity, hist_movie_ids, hist_genre_ids, user_table, gender_table, age_table, occup_table, zip_table, year_table, month_table, weekday_table, hour_table, item_table, genre_table, act_W, act_b, W1, b1, W2, b2):
    raise NotImplementedError("write your pallas kernel here")


# speedup vs baseline: 10.0699x; 10.0699x over previous
import jax
import jax.numpy as jnp
from jax import lax
from jax.experimental import pallas as pl
from jax.experimental.pallas import tpu as pltpu
from jax.experimental.pallas import tpu_sc as plsc

B = 4096
L = 200
NC = 2
NS = 16
NW = NC * NS
RPW = B // NW
NBUF = 4
C0, C1 = 104, 96
ACC = 4


def _sc_body(uid_h, zipc_h, mv_h, gn_h, user_t, zip_t, item_t,
             user_o, zip_o, isum_o, gcnt_o,
             uid_v, zipc_v, mv_v, gn_v, urows_v, zrows_v, ring_v,
             isum_v, gcnt_v, sem_u, sem_z, sem0, sem1, sem2, sem3):
    sems = (sem0, sem1, sem2, sem3)
    wid = lax.axis_index("s") * NC + lax.axis_index("c")
    base = wid * RPW

    pltpu.sync_copy(uid_h.at[pl.ds(base, RPW)], uid_v)
    pltpu.sync_copy(zipc_h.at[pl.ds(base, RPW)], zipc_v)
    pltpu.sync_copy(mv_h.at[pl.ds(base, RPW)], mv_v)
    pltpu.sync_copy(gn_h.at[pl.ds(base, RPW)], gn_v)

    cu = pltpu.async_copy(user_t.at[uid_v], urows_v, sem_u)
    cz = pltpu.async_copy(zip_t.at[zipc_v], zrows_v, sem_z)

    def issue(b, slot):
        pltpu.make_async_copy(item_t.at[mv_v.at[b, pl.ds(0, C0)]],
                              ring_v.at[slot, pl.ds(0, C0)], sems[slot]).start()
        pltpu.make_async_copy(item_t.at[mv_v.at[b, pl.ds(C0, C1)]],
                              ring_v.at[slot, pl.ds(C0, C1)], sems[slot]).start()

    for i in range(NBUF):
        issue(i, i)

    lanes = lax.iota(jnp.int32, 16)
    ones16 = jnp.ones((16,), jnp.float32)
    zeros16 = jnp.zeros((16,), jnp.float32)

    def zrow(r, c):
        gcnt_v[r, pl.ds(0, 16)] = zeros16
        gcnt_v[r, pl.ds(16, 16)] = zeros16
        return c
    lax.fori_loop(0, RPW, zrow, 0)

    def g_outer(g, c):
        rows = lanes + g * 16

        def g_inner(l, c2):
            cols = jnp.zeros((16,), jnp.int32) + l
            gids = plsc.load_gather(gn_v, [rows, cols])
            plsc.addupdate_scatter(gcnt_v, [rows, gids], ones16)
            return c2
        return lax.fori_loop(0, L, g_inner, c)
    lax.fori_loop(0, RPW // 16, g_outer, 0)

    def outer(k, c):
        bb = k * NBUF
        for i in range(NBUF):
            b = bb + i
            pltpu.make_async_copy(item_t.at[pl.ds(0, L)], ring_v.at[i],
                                  sems[i]).wait()

            def acc_body(j, accs, i=i):
                r0 = j * 8
                new = list(accs)
                for r in range(8):
                    lo = ring_v[i, r0 + r, pl.ds(0, 16)]
                    hi = ring_v[i, r0 + r, pl.ds(16, 16)]
                    a = r % ACC
                    new[2 * a] = new[2 * a] + lo
                    new[2 * a + 1] = new[2 * a + 1] + hi
                return tuple(new)

            accs = lax.fori_loop(0, L // 8, acc_body, (zeros16,) * (2 * ACC))
            isum_v[b, pl.ds(0, 16)] = accs[0] + accs[2] + accs[4] + accs[6]
            isum_v[b, pl.ds(16, 16)] = accs[1] + accs[3] + accs[5] + accs[7]
            nb = b + NBUF

            @pl.when(nb < RPW)
            def _(nb=nb, i=i):
                issue(nb, i)
        return c
    lax.fori_loop(0, RPW // NBUF, outer, 0)

    cu.wait()
    cz.wait()
    pltpu.sync_copy(urows_v, user_o.at[pl.ds(base, RPW)])
    pltpu.sync_copy(zrows_v, zip_o.at[pl.ds(base, RPW)])
    pltpu.sync_copy(isum_v, isum_o.at[pl.ds(base, RPW)])
    pltpu.sync_copy(gcnt_v, gcnt_o.at[pl.ds(base, RPW)])


def _tc_body(gender_r, age_r, occup_r, year_r, month_r, hour_r, weekday_r,
             act_r, gender_t_r, age_t_r, occup_t_r, year_t_r, month_t_r,
             hour_t_r, weekday_t_r, genre_t_r, act_W_r, act_b_r,
             W1_r, b1_r, W2_r, b2_r,
             user_r, zip_r, isum_r, gcnt_r, out_r):
    f32 = jnp.float32

    def oh(idx_col, K, tbl, zero_row0=False):
        cols = lax.broadcasted_iota(jnp.int32, (B, K), 1)
        m = (idx_col == cols).astype(f32)
        if zero_row0:
            ri = lax.broadcasted_iota(jnp.int32, tbl.shape, 0)
            tbl = jnp.where(ri == 0, 0.0, tbl)
        return m @ tbl

    gender_v = oh(gender_r[...], 4, gender_t_r[...])
    age_v = oh(age_r[...], 8, age_t_r[...])
    occup_v = oh(occup_r[...], 32, occup_t_r[...])
    year_v = oh(year_r[...], 16, year_t_r[...])
    month_v = oh(month_r[...], 12, month_t_r[...])
    hour_v = oh(hour_r[...], 25, hour_t_r[...], True)
    weekday_v = oh(weekday_r[...], 8, weekday_t_r[...], True)
    act_v = jnp.maximum(act_r[...] @ act_W_r[...] + act_b_r[...], 0.0)
    seq_v = (isum_r[...] + gcnt_r[...] @ genre_t_r[...]) * (1.0 / L)
    comb = jnp.concatenate(
        [user_r[...], gender_v, age_v, occup_v, zip_r[...],
         year_v, month_v, hour_v, weekday_v, act_v, seq_v], axis=1)
    h = jnp.maximum(comb @ W1_r[...] + b1_r[...], 0.0)
    out_r[...] = h @ W2_r[...] + b2_r[...]


def kernel(user_id, gender, age, occup, zip, year, month, hour, weekday,
           user_activity, hist_movie_ids, hist_genre_ids,
           user_table, gender_table, age_table, occup_table, zip_table,
           year_table, month_table, weekday_table, hour_table,
           item_table, genre_table, act_W, act_b, W1, b1, W2, b2):
    i32 = jnp.int32
    f32 = jnp.float32
    sds = jax.ShapeDtypeStruct

    sc = pl.kernel(
        _sc_body,
        out_type=[sds((B, 32), f32)] * 4,
        mesh=plsc.VectorSubcoreMesh(core_axis_name="c", subcore_axis_name="s"),
        compiler_params=pltpu.CompilerParams(use_tc_tiling_on_sc=False,
                                             needs_layout_passes=False),
        scratch_types=[
            pltpu.VMEM((RPW,), i32),
            pltpu.VMEM((RPW,), i32),
            pltpu.VMEM((RPW, L), i32),
            pltpu.VMEM((RPW, L), i32),
            pltpu.VMEM((RPW, 32), f32),
            pltpu.VMEM((RPW, 32), f32),
            pltpu.VMEM((NBUF, L, 32), f32),
            pltpu.VMEM((RPW, 32), f32),
            pltpu.VMEM((RPW, 32), f32),
            pltpu.SemaphoreType.DMA,
            pltpu.SemaphoreType.DMA,
            pltpu.SemaphoreType.DMA,
            pltpu.SemaphoreType.DMA,
            pltpu.SemaphoreType.DMA,
            pltpu.SemaphoreType.DMA,
        ],
    )
    user_rows, zip_rows, isum, gcnt = sc(
        user_id.astype(i32), zip.astype(i32),
        hist_movie_ids.astype(i32), hist_genre_ids.astype(i32),
        user_table, zip_table, item_table)

    out = pl.pallas_call(
        _tc_body,
        out_shape=sds((B, 128), f32),
    )(gender.astype(i32).reshape(B, 1), age.astype(i32).reshape(B, 1),
      occup.astype(i32).reshape(B, 1), year.astype(i32).reshape(B, 1),
      month.astype(i32).reshape(B, 1), hour.astype(i32).reshape(B, 1),
      weekday.astype(i32).reshape(B, 1),
      user_activity, gender_table, age_table, occup_table, year_table,
      month_table, hour_table, weekday_table, genre_table,
      act_W, act_b.reshape(1, 16), W1, b1.reshape(1, 256),
      W2, b2.reshape(1, 128),
      user_rows, zip_rows, isum, gcnt)
    return out


# speedup vs baseline: 10.1182x; 1.0048x over previous
import jax
import jax.numpy as jnp
from jax import lax
from jax.experimental import pallas as pl
from jax.experimental.pallas import tpu as pltpu
from jax.experimental.pallas import tpu_sc as plsc

B = 4096
L = 200
NC = 2
NS = 16
NW = NC * NS
RPW = B // NW
NBUF = 4
C0, C1 = 104, 96
ACC = 4


def _sc_body(uid_h, zipc_h, mv_h, gn_h, user_t, zip_t, item_t,
             user_o, zip_o, isum_o, gcnt_o,
             uid_v, zipc_v, mv_v, gn_v, urows_v, zrows_v, ring_v,
             isum_v, gcnt_v, sem_u, sem_z, sem0, sem1, sem2, sem3):
    sems = (sem0, sem1, sem2, sem3)
    wid = lax.axis_index("s") * NC + lax.axis_index("c")
    base = wid * RPW

    pltpu.sync_copy(uid_h.at[pl.ds(base, RPW)], uid_v)
    pltpu.sync_copy(zipc_h.at[pl.ds(base, RPW)], zipc_v)
    pltpu.sync_copy(mv_h.at[pl.ds(base * L, RPW * L)], mv_v)
    pltpu.sync_copy(gn_h.at[pl.ds(base * L, RPW * L)], gn_v)

    cu = pltpu.async_copy(user_t.at[uid_v], urows_v, sem_u)
    cz = pltpu.async_copy(zip_t.at[zipc_v], zrows_v, sem_z)

    def issue(b, slot):
        pltpu.make_async_copy(item_t.at[mv_v.at[pl.ds(b * L, C0)]],
                              ring_v.at[slot, pl.ds(0, C0)], sems[slot]).start()
        pltpu.make_async_copy(item_t.at[mv_v.at[pl.ds(b * L + C0, C1)]],
                              ring_v.at[slot, pl.ds(C0, C1)], sems[slot]).start()

    for i in range(NBUF):
        issue(i, i)

    lanes = lax.iota(jnp.int32, 16)
    ones16 = jnp.ones((16,), jnp.float32)
    zeros16 = jnp.zeros((16,), jnp.float32)

    def zrow(r, c):
        gcnt_v[r, pl.ds(0, 16)] = zeros16
        gcnt_v[r, pl.ds(16, 16)] = zeros16
        return c
    lax.fori_loop(0, RPW, zrow, 0)

    def g_outer(g, c):
        rows = lanes + g * 16

        def g_inner(l, c2):
            gids = plsc.load_gather(gn_v, [rows * L + l])
            plsc.addupdate_scatter(gcnt_v, [rows, gids], ones16)
            return c2
        return lax.fori_loop(0, L, g_inner, c)
    lax.fori_loop(0, RPW // 16, g_outer, 0)

    def outer(k, c):
        bb = k * NBUF
        for i in range(NBUF):
            b = bb + i
            pltpu.make_async_copy(item_t.at[pl.ds(0, L)], ring_v.at[i],
                                  sems[i]).wait()

            def acc_body(j, accs, i=i):
                r0 = j * 8
                new = list(accs)
                for r in range(8):
                    lo = ring_v[i, r0 + r, pl.ds(0, 16)]
                    hi = ring_v[i, r0 + r, pl.ds(16, 16)]
                    a = r % ACC
                    new[2 * a] = new[2 * a] + lo
                    new[2 * a + 1] = new[2 * a + 1] + hi
                return tuple(new)

            accs = lax.fori_loop(0, L // 8, acc_body, (zeros16,) * (2 * ACC))
            isum_v[b, pl.ds(0, 16)] = accs[0] + accs[2] + accs[4] + accs[6]
            isum_v[b, pl.ds(16, 16)] = accs[1] + accs[3] + accs[5] + accs[7]
            nb = b + NBUF

            @pl.when(nb < RPW)
            def _(nb=nb, i=i):
                issue(nb, i)
        return c
    lax.fori_loop(0, RPW // NBUF, outer, 0)

    cu.wait()
    cz.wait()
    pltpu.sync_copy(urows_v, user_o.at[pl.ds(base, RPW)])
    pltpu.sync_copy(zrows_v, zip_o.at[pl.ds(base, RPW)])
    pltpu.sync_copy(isum_v, isum_o.at[pl.ds(base, RPW)])
    pltpu.sync_copy(gcnt_v, gcnt_o.at[pl.ds(base, RPW)])


def _tc_body(gender_r, age_r, occup_r, year_r, month_r, hour_r, weekday_r,
             act_r, gender_t_r, age_t_r, occup_t_r, year_t_r, month_t_r,
             hour_t_r, weekday_t_r, genre_t_r, act_W_r, act_b_r,
             W1_r, b1_r, W2_r, b2_r,
             user_r, zip_r, isum_r, gcnt_r, out_r):
    f32 = jnp.float32

    def oh(idx_col, K, tbl, zero_row0=False):
        cols = lax.broadcasted_iota(jnp.int32, (B, K), 1)
        m = (idx_col == cols).astype(f32)
        if zero_row0:
            ri = lax.broadcasted_iota(jnp.int32, tbl.shape, 0)
            tbl = jnp.where(ri == 0, 0.0, tbl)
        return m @ tbl

    gender_v = oh(gender_r[...], 4, gender_t_r[...])
    age_v = oh(age_r[...], 8, age_t_r[...])
    occup_v = oh(occup_r[...], 32, occup_t_r[...])
    year_v = oh(year_r[...], 16, year_t_r[...])
    month_v = oh(month_r[...], 12, month_t_r[...])
    hour_v = oh(hour_r[...], 25, hour_t_r[...], True)
    weekday_v = oh(weekday_r[...], 8, weekday_t_r[...], True)
    act_v = jnp.maximum(act_r[...] @ act_W_r[...] + act_b_r[...], 0.0)
    seq_v = (isum_r[...] + gcnt_r[...] @ genre_t_r[...]) * (1.0 / L)
    comb = jnp.concatenate(
        [user_r[...], gender_v, age_v, occup_v, zip_r[...],
         year_v, month_v, hour_v, weekday_v, act_v, seq_v], axis=1)
    h = jnp.maximum(comb @ W1_r[...] + b1_r[...], 0.0)
    out_r[...] = h @ W2_r[...] + b2_r[...]


def kernel(user_id, gender, age, occup, zip, year, month, hour, weekday,
           user_activity, hist_movie_ids, hist_genre_ids,
           user_table, gender_table, age_table, occup_table, zip_table,
           year_table, month_table, weekday_table, hour_table,
           item_table, genre_table, act_W, act_b, W1, b1, W2, b2):
    i32 = jnp.int32
    f32 = jnp.float32
    sds = jax.ShapeDtypeStruct

    sc = pl.kernel(
        _sc_body,
        out_type=[sds((B, 32), f32)] * 4,
        mesh=plsc.VectorSubcoreMesh(core_axis_name="c", subcore_axis_name="s"),
        compiler_params=pltpu.CompilerParams(use_tc_tiling_on_sc=False,
                                             needs_layout_passes=False),
        scratch_types=[
            pltpu.VMEM((RPW,), i32),
            pltpu.VMEM((RPW,), i32),
            pltpu.VMEM((RPW * L,), i32),
            pltpu.VMEM((RPW * L,), i32),
            pltpu.VMEM((RPW, 32), f32),
            pltpu.VMEM((RPW, 32), f32),
            pltpu.VMEM((NBUF, L, 32), f32),
            pltpu.VMEM((RPW, 32), f32),
            pltpu.VMEM((RPW, 32), f32),
            pltpu.SemaphoreType.DMA,
            pltpu.SemaphoreType.DMA,
            pltpu.SemaphoreType.DMA,
            pltpu.SemaphoreType.DMA,
            pltpu.SemaphoreType.DMA,
            pltpu.SemaphoreType.DMA,
        ],
    )
    user_rows, zip_rows, isum, gcnt = sc(
        user_id.astype(i32), zip.astype(i32),
        hist_movie_ids.astype(i32).reshape(B * L),
        hist_genre_ids.astype(i32).reshape(B * L),
        user_table, zip_table, item_table)

    out = pl.pallas_call(
        _tc_body,
        out_shape=sds((B, 128), f32),
    )(gender.astype(i32).reshape(B, 1), age.astype(i32).reshape(B, 1),
      occup.astype(i32).reshape(B, 1), year.astype(i32).reshape(B, 1),
      month.astype(i32).reshape(B, 1), hour.astype(i32).reshape(B, 1),
      weekday.astype(i32).reshape(B, 1),
      user_activity, gender_table, age_table, occup_table, year_table,
      month_table, hour_table, weekday_table, genre_table,
      act_W, act_b.reshape(1, 16), W1, b1.reshape(1, 256),
      W2, b2.reshape(1, 128),
      user_rows, zip_rows, isum, gcnt)
    return out
